# initial kernel scaffold (unmeasured)
import jax
import jax.numpy as jnp
from jax import lax
from jax.experimental import pallas as pl
from jax.experimental.pallas import tpu as pltpu


def kernel(
    x,
):
    def body(*refs):
        pass

    out_shape = jax.ShapeDtypeStruct(..., jnp.float32)
    return pl.pallas_call(body, out_shape=out_shape)(...)


# baseline (device time: 30980 ns/iter reference)
import jax
import jax.numpy as jnp
from jax import lax
from jax.experimental import pallas as pl
from jax.experimental.pallas import tpu as pltpu


def kernel(x):
    m, n = x.shape

    def body(x_ref, out_ref, comm_ref, send_sems, recv_sems):
        my_x = lax.axis_index("x")
        my_y = lax.axis_index("y")
        x_nbr = (1 - my_x, my_y)
        y_nbr = (my_x, 1 - my_y)

        barrier_sem = pltpu.get_barrier_semaphore()
        for nbr in (x_nbr, y_nbr):
            pl.semaphore_signal(
                barrier_sem, inc=1,
                device_id=nbr, device_id_type=pl.DeviceIdType.MESH,
            )
        pl.semaphore_wait(barrier_sem, 2)

        rdma1 = pltpu.make_async_remote_copy(
            src_ref=x_ref,
            dst_ref=comm_ref.at[0],
            send_sem=send_sems.at[0],
            recv_sem=recv_sems.at[0],
            device_id=x_nbr,
            device_id_type=pl.DeviceIdType.MESH,
        )
        rdma1.start()
        rdma1.wait()
        out_ref[:, :] = x_ref[:, :] + comm_ref[0, :, :]

        rdma2 = pltpu.make_async_remote_copy(
            src_ref=out_ref,
            dst_ref=comm_ref.at[1],
            send_sem=send_sems.at[1],
            recv_sem=recv_sems.at[1],
            device_id=y_nbr,
            device_id_type=pl.DeviceIdType.MESH,
        )
        rdma2.start()
        rdma2.wait()
        out_ref[:, :] = out_ref[:, :] + comm_ref[1, :, :]

    return pl.pallas_call(
        body,
        out_shape=jax.ShapeDtypeStruct((m, n), x.dtype),
        in_specs=[pl.BlockSpec(memory_space=pltpu.VMEM)],
        out_specs=pl.BlockSpec(memory_space=pltpu.VMEM),
        scratch_shapes=[
            pltpu.VMEM((2, m, n), x.dtype),
            pltpu.SemaphoreType.DMA((2,)),
            pltpu.SemaphoreType.DMA((2,)),
        ],
        compiler_params=pltpu.CompilerParams(collective_id=0),
    )(x)


# device time: 19746 ns/iter; 1.5689x vs baseline; 1.5689x over previous
import jax
import jax.numpy as jnp
from jax import lax
from jax.experimental import pallas as pl
from jax.experimental.pallas import tpu as pltpu


def kernel(x):
    m, n = x.shape
    h = m // 2

    def body(x_ref, out_ref, comm_ref, send_sems, recv_sems):
        my_x = lax.axis_index("x")
        my_y = lax.axis_index("y")
        x_nbr = (1 - my_x, my_y)
        y_nbr = (my_x, 1 - my_y)

        barrier_sem = pltpu.get_barrier_semaphore()
        for nbr in (x_nbr, y_nbr):
            pl.semaphore_signal(
                barrier_sem, inc=1,
                device_id=nbr, device_id_type=pl.DeviceIdType.MESH,
            )
        pl.semaphore_wait(barrier_sem, 2)

        def exchange(src, slot, nbr):
            return pltpu.make_async_remote_copy(
                src_ref=src,
                dst_ref=comm_ref.at[slot],
                send_sem=send_sems.at[slot],
                recv_sem=recv_sems.at[slot],
                device_id=nbr,
                device_id_type=pl.DeviceIdType.MESH,
            )

        a1 = exchange(x_ref.at[pl.ds(0, h), :], 0, x_nbr)
        b1 = exchange(x_ref.at[pl.ds(h, h), :], 1, y_nbr)
        a1.start()
        b1.start()

        a1.wait()
        out_ref[pl.ds(0, h), :] = x_ref[pl.ds(0, h), :] + comm_ref[0, :, :]
        a2 = exchange(out_ref.at[pl.ds(0, h), :], 2, y_nbr)
        a2.start()

        b1.wait()
        out_ref[pl.ds(h, h), :] = x_ref[pl.ds(h, h), :] + comm_ref[1, :, :]
        b2 = exchange(out_ref.at[pl.ds(h, h), :], 3, x_nbr)
        b2.start()

        a2.wait()
        out_ref[pl.ds(0, h), :] = out_ref[pl.ds(0, h), :] + comm_ref[2, :, :]
        b2.wait()
        out_ref[pl.ds(h, h), :] = out_ref[pl.ds(h, h), :] + comm_ref[3, :, :]

    return pl.pallas_call(
        body,
        out_shape=jax.ShapeDtypeStruct((m, n), x.dtype),
        in_specs=[pl.BlockSpec(memory_space=pltpu.VMEM)],
        out_specs=pl.BlockSpec(memory_space=pltpu.VMEM),
        scratch_shapes=[
            pltpu.VMEM((4, h, n), x.dtype),
            pltpu.SemaphoreType.DMA((4,)),
            pltpu.SemaphoreType.DMA((4,)),
        ],
        compiler_params=pltpu.CompilerParams(collective_id=0),
    )(x)


# device time: 18493 ns/iter; 1.6752x vs baseline; 1.0678x over previous
import jax
import jax.numpy as jnp
from jax import lax
from jax.experimental import pallas as pl
from jax.experimental.pallas import tpu as pltpu

CHUNKS = 2


def kernel(x):
    m, n = x.shape
    c = m // (2 * CHUNKS)
    n_slots = 4 * CHUNKS

    def body(x_ref, out_ref, comm_ref, send_sems, recv_sems):
        my_x = lax.axis_index("x")
        my_y = lax.axis_index("y")
        x_nbr = (1 - my_x, my_y)
        y_nbr = (my_x, 1 - my_y)

        barrier_sem = pltpu.get_barrier_semaphore()
        for nbr in (x_nbr, y_nbr):
            pl.semaphore_signal(
                barrier_sem, inc=1,
                device_id=nbr, device_id_type=pl.DeviceIdType.MESH,
            )
        pl.semaphore_wait(barrier_sem, 2)

        def exchange(src, slot, nbr):
            return pltpu.make_async_remote_copy(
                src_ref=src,
                dst_ref=comm_ref.at[slot],
                send_sem=send_sems.at[slot],
                recv_sem=recv_sems.at[slot],
                device_id=nbr,
                device_id_type=pl.DeviceIdType.MESH,
            )

        p1 = []
        for k in range(CHUNKS):
            p1.append((exchange(x_ref.at[pl.ds(k * c, c), :], k, x_nbr),
                       k * c, k))
            p1.append((exchange(
                x_ref.at[pl.ds(m // 2 + k * c, c), :], CHUNKS + k, y_nbr),
                m // 2 + k * c, CHUNKS + k))
        for rdma, _, _ in p1:
            rdma.start()

        p2 = []
        for rdma, row, slot in p1:
            rdma.wait()
            out_ref[pl.ds(row, c), :] = (
                x_ref[pl.ds(row, c), :] + comm_ref[slot, :, :]
            )
            nbr2 = y_nbr if slot < CHUNKS else x_nbr
            rdma2 = exchange(out_ref.at[pl.ds(row, c), :], 2 * CHUNKS + slot,
                             nbr2)
            rdma2.start()
            p2.append((rdma2, row, 2 * CHUNKS + slot))

        for rdma2, row, slot in p2:
            rdma2.wait()
            out_ref[pl.ds(row, c), :] = (
                out_ref[pl.ds(row, c), :] + comm_ref[slot, :, :]
            )

    return pl.pallas_call(
        body,
        out_shape=jax.ShapeDtypeStruct((m, n), x.dtype),
        in_specs=[pl.BlockSpec(memory_space=pltpu.VMEM)],
        out_specs=pl.BlockSpec(memory_space=pltpu.VMEM),
        scratch_shapes=[
            pltpu.VMEM((n_slots, c, n), x.dtype),
            pltpu.SemaphoreType.DMA((n_slots,)),
            pltpu.SemaphoreType.DMA((n_slots,)),
        ],
        compiler_params=pltpu.CompilerParams(collective_id=0),
    )(x)
